# baseline (speedup 1.0000x reference)
import functools

import jax
import jax.numpy as jnp
from jax import lax
from jax.experimental import pallas as pl
from jax.experimental.pallas import tpu as pltpu
from jax.experimental.pallas import tpu_sc as plsc

NC = 2
NS = 16
NW = NC * NS
L = 16
C = 40
GS = (0, 16, 24)
CS = 80
ZR = 128


def _prep_body(h_ref, wa_ref, wb_ref, be1_ref, we2_ref, wc1_ref, be2_ref,
               bc1_ref, a_ref, b_ref, wp_ref, bp_ref):
  hb = h_ref[...]
  a_ref[...] = jnp.dot(hb, wa_ref[...].T, preferred_element_type=jnp.float32) + be1_ref[...]
  b_ref[...] = jnp.dot(hb, wb_ref[...].T, preferred_element_type=jnp.float32)
  wp_ref[...] = jnp.dot(we2_ref[...].T, wc1_ref[...].T,
                        preferred_element_type=jnp.float32)
  bp_ref[...] = jnp.dot(be2_ref[...], wc1_ref[...].T,
                        preferred_element_type=jnp.float32) + bc1_ref[...]


def _edge_mlp_body(r_ref, wp_ref, bp_ref, wc2_ref, cw_ref):
  t = jnp.maximum(
      jnp.dot(r_ref[...], wp_ref[...], preferred_element_type=jnp.float32)
      + bp_ref[...], 0.0)
  cw_ref[...] = jnp.sum(t * wc2_ref[...], axis=1, keepdims=True)


def _node_body(h_ref, agg0_ref, agg1_ref, cux_ref, cuy_ref, cuz_ref, cd_ref,
               coords_ref, we2_ref, be2_ref, wn1a_ref, wn1b_ref, bn1_ref,
               wn2_ref, bn2_ref, hnew_ref, cout_ref):
  bn = h_ref.shape[0]
  rsum = agg0_ref[...] + agg1_ref[...]
  deg = jnp.sum(cd_ref[...].reshape(NC, bn).T, axis=1, keepdims=True)
  aggm = jnp.dot(rsum, we2_ref[...].T, preferred_element_type=jnp.float32) \
      + deg * be2_ref[...]
  pre = (jnp.dot(h_ref[...], wn1a_ref[...].T, preferred_element_type=jnp.float32)
         + jnp.dot(aggm, wn1b_ref[...].T, preferred_element_type=jnp.float32)
         + bn1_ref[...])
  hnew_ref[...] = jnp.dot(jnp.maximum(pre, 0.0), wn2_ref[...].T,
                          preferred_element_type=jnp.float32) + bn2_ref[...]
  ux = jnp.sum(cux_ref[...].reshape(NC, bn).T, axis=1, keepdims=True)
  uy = jnp.sum(cuy_ref[...].reshape(NC, bn).T, axis=1, keepdims=True)
  uz = jnp.sum(cuz_ref[...].reshape(NC, bn).T, axis=1, keepdims=True)
  cout_ref[...] = coords_ref[...] + jnp.concatenate([ux, uy, uz], axis=1)


def _zero_vmem2(ref, nrows, ncols):
  z = jnp.zeros((L,), jnp.float32)
  def body(i, _):
    for j in range(ncols // L):
      ref[i, pl.ds(j * L, L)] = z
    return 0
  lax.fori_loop(0, nrows, body, 0)


def _tile_rows(n):
  rpt = (-(-n // NS) + 7) // 8 * 8
  last = n - (NS - 1) * rpt
  assert last > 0
  return rpt, last


def _fill_rows(zsrc, nz, dst, start, cnt):
  for t in range(0, cnt, nz):
    c = min(nz, cnt - t)
    pltpu.sync_copy(zsrc.at[pl.ds(0, c)], dst.at[pl.ds(start + t, c)])


def _per_tile_ranges(sid, n, fn):
  rpt, last = _tile_rows(n)
  @pl.when(sid < NS - 1)
  def _():
    fn(sid * rpt, rpt)
  @pl.when(sid == NS - 1)
  def _():
    fn((NS - 1) * rpt, last)


def _edge_sc_body(a_hbm, b_hbm, cx_hbm, cy_hbm, cz_hbm, row_hbm, col_hbm,
                  w3_hbm, rh_hbm, rx_hbm, ry_hbm, rz_hbm, aggp_hbm,
                  rowv0, rowv1, colv0, colv1, arows0, arows1, brows0, brows1,
                  xr0, xr1, yr0, yr1, zr0, zr1, xc0, xc1, yc0, yc1, zc0, zc1,
                  rbuf0, rbuf1, rxb0, rxb1, ryb0, ryb1,
                  rzb0, rzb1, w3v, sq_smem, aggsh,
                  isem0, isem1, gsem0, gsem1, wsem0, wsem1):
  n = aggsh.shape[0]
  d = arows0.shape[1]
  cid = lax.axis_index("c")
  sid = lax.axis_index("s")
  wid = sid * NC + cid
  e = row_hbm.shape[0]
  kpt = e // (NW * C)
  tile_e0 = wid * kpt * C

  rowv = (rowv0, rowv1); colv = (colv0, colv1)
  arows = (arows0, arows1); brows = (brows0, brows1)
  xr = (xr0, xr1); yr = (yr0, yr1); zr = (zr0, zr1)
  xc = (xc0, xc1); yc = (yc0, yc1); zc = (zc0, zc1)
  rbuf = (rbuf0, rbuf1)
  rxb = (rxb0, rxb1); ryb = (ryb0, ryb1); rzb = (rzb0, rzb1)
  isem = (isem0, isem1); gsem = (gsem0, gsem1); wsem = (wsem0, wsem1)

  pltpu.sync_copy(w3_hbm, w3v)
  w3vecs = [w3v[pl.ds(j * L, L)] for j in range(d // L)]

  _zero_vmem2(rbuf0, C, d)
  _per_tile_ranges(sid, n, lambda s, c: _fill_rows(rbuf0, C, aggsh, s, c))
  plsc.subcore_barrier()

  def idescs(k, b):
    return [(row_hbm.at[pl.ds(tile_e0 + k * C, C)], rowv[b]),
            (col_hbm.at[pl.ds(tile_e0 + k * C, C)], colv[b])]

  def gdescs(b):
    return [(a_hbm.at[rowv[b]], arows[b]),
            (b_hbm.at[colv[b]], brows[b]),
            (cx_hbm.at[rowv[b]], xr[b]),
            (cy_hbm.at[rowv[b]], yr[b]),
            (cz_hbm.at[rowv[b]], zr[b]),
            (cx_hbm.at[colv[b]], xc[b]),
            (cy_hbm.at[colv[b]], yc[b]),
            (cz_hbm.at[colv[b]], zc[b])]

  def wdescs(k, b):
    base = tile_e0 + k * C
    return [(rbuf[b], rh_hbm.at[pl.ds(base, C)]),
            (rxb[b], rx_hbm.at[pl.ds(base, C)]),
            (ryb[b], ry_hbm.at[pl.ds(base, C)]),
            (rzb[b], rz_hbm.at[pl.ds(base, C)])]

  def process(k, b):
    @pl.when(k >= 2)
    def _():
      for src, dst in wdescs(k, b):
        pltpu.make_async_copy(src, dst, wsem[b]).wait()
    for src, dst in gdescs(b):
      pltpu.make_async_copy(src, dst, gsem[b]).wait()
    @pl.when((k >= 1) & (k < kpt - 1))
    def _():
      for src, dst in idescs(k + 1, b ^ 1):
        pltpu.make_async_copy(src, dst, isem[b ^ 1]).wait()
    @pl.when(k < kpt - 1)
    def _():
      for src, dst in gdescs(b ^ 1):
        pltpu.async_copy(src, dst, gsem[b ^ 1])

    for g in GS:
      gs = pl.ds(g, L)
      relx = xr[b][gs] - xc[b][gs]
      rely = yr[b][gs] - yc[b][gs]
      relz = zr[b][gs] - zc[b][gs]
      rxb[b][gs] = relx
      ryb[b][gs] = rely
      rzb[b][gs] = relz
      sqv = relx * relx + rely * rely + relz * relz
      for lane in range(L):
        sq_smem[g + lane] = sqv[lane]

    def edge_body(e2, _):
      for u in range(2):
        ei = e2 * 2 + u
        sq = sq_smem[ei]
        for j in range(d // L):
          va = arows[b][ei, pl.ds(j * L, L)]
          vb = brows[b][ei, pl.ds(j * L, L)]
          rbuf[b][ei, pl.ds(j * L, L)] = jnp.maximum(
              va + vb + sq * w3vecs[j], 0.0)
      return 0
    lax.fori_loop(0, C // 2, edge_body, 0)

    for src, dst in wdescs(k, b):
      pltpu.async_copy(src, dst, wsem[b])
    pltpu.sync_copy(rbuf[b], aggsh.at[rowv[b]], add=True)
    @pl.when(k < kpt - 2)
    def _():
      for src, dst in idescs(k + 2, b):
        pltpu.async_copy(src, dst, isem[b])

  for src, dst in idescs(0, 0):
    pltpu.sync_copy(src, dst)
  for src, dst in idescs(1, 1):
    pltpu.sync_copy(src, dst)
  for src, dst in gdescs(0):
    pltpu.async_copy(src, dst, gsem[0])

  def pair_body(p, _):
    for b in range(2):
      process(p * 2 + b, b)
    return 0
  assert True
  lax.fori_loop(0, kpt // 2, pair_body, 0)
  if kpt % 2:
    process(kpt - 1, (kpt - 1) % 2)

  for b in range(2):
    klast = kpt - 1 - ((kpt - 1 + b) % 2)
    for src, dst in wdescs(klast, b):
      pltpu.make_async_copy(src, dst, wsem[b]).wait()

  plsc.subcore_barrier()
  _per_tile_ranges(
      sid, n,
      lambda s, c: pltpu.sync_copy(aggsh.at[pl.ds(s, c)],
                                   aggp_hbm.at[cid, pl.ds(s, c)]))


def _coord_sc_body(cw_hbm, rx_hbm, ry_hbm, rz_hbm, row4_hbm,
                   cupx_hbm, cupy_hbm, cupz_hbm, cdeg_hbm,
                   rowa, cwa, rxa, rya, rza, pxa, pya, pza, onesb, zb,
                   cshx, cshy, cshz, cshd, ssem):
  n = cshx.shape[0]
  cid = lax.axis_index("c")
  sid = lax.axis_index("s")
  wid = sid * NC + cid
  kpt = rowa.shape[0]
  ept = kpt * CS
  tile_e0 = wid * ept

  one = jnp.full((L,), 1.0, jnp.float32)
  for t in range(CS // L):
    onesb[pl.ds(t * L, L)] = one
  for t in range(ZR // L):
    zb[pl.ds(t * L, L)] = jnp.zeros((L,), jnp.float32)
  for sh in (cshx, cshy, cshz, cshd):
    _per_tile_ranges(sid, n, lambda s, c, sh=sh: _fill_rows(zb, ZR, sh, s, c))

  pltpu.sync_copy(row4_hbm.at[wid], rowa)
  pltpu.sync_copy(cw_hbm.at[pl.ds(tile_e0, ept)], cwa)
  pltpu.sync_copy(rx_hbm.at[pl.ds(tile_e0, ept)], rxa)
  pltpu.sync_copy(ry_hbm.at[pl.ds(tile_e0, ept)], rya)
  pltpu.sync_copy(rz_hbm.at[pl.ds(tile_e0, ept)], rza)

  def prod_body(i, _):
    s = pl.ds(i * L, L)
    w = cwa[s]
    pxa[s] = w * rxa[s]
    pya[s] = w * rya[s]
    pza[s] = w * rza[s]
    return 0
  lax.fori_loop(0, ept // L, prod_body, 0)
  plsc.subcore_barrier()

  def sdescs(k):
    es = pl.ds(k * CS, CS)
    return [(pxa.at[es], cshx.at[rowa.at[k, 0]]),
            (pya.at[es], cshy.at[rowa.at[k, 0]]),
            (pza.at[es], cshz.at[rowa.at[k, 0]]),
            (onesb, cshd.at[rowa.at[k, 0]])]

  def chunk_body(k, _):
    for src, dst in sdescs(k):
      pltpu.async_copy(src, dst, ssem, add=True)
    @pl.when(k >= 1)
    def _():
      for src, dst in sdescs(k - 1):
        pltpu.make_async_copy(src, dst, ssem).wait()
    return 0
  lax.fori_loop(0, kpt, chunk_body, 0)
  for src, dst in sdescs(kpt - 1):
    pltpu.make_async_copy(src, dst, ssem).wait()

  plsc.subcore_barrier()
  def _staged_out(sh, out, s, c):
    for t in range(0, c, ZR):
      cc = min(ZR, c - t)
      pltpu.sync_copy(sh.at[pl.ds(s + t, cc)], pxa.at[pl.ds(0, cc)])
      pltpu.sync_copy(pxa.at[pl.ds(0, cc)],
                      out.at[pl.ds(cid * n + s + t, cc)])
  for sh, out in ((cshx, cupx_hbm), (cshy, cupy_hbm), (cshz, cupz_hbm),
                  (cshd, cdeg_hbm)):
    _per_tile_ranges(
        sid, n,
        lambda s, c, sh=sh, out=out: _staged_out(sh, out, s, c))


def kernel(h, coords, edge_index, We1, be1, We2, be2, Wn1, bn1, Wn2, bn2,
           Wc1, bc1, Wc2):
  n, d = h.shape
  e = edge_index.shape[1]
  assert e % (NW * C) == 0 and e % (NW * CS) == 0 and d % L == 0
  kpts = e // (NW * CS)

  row = edge_index[0]
  col = edge_index[1]
  row4 = row.reshape(NW, kpts, 1, CS)
  cx = coords[:, 0]
  cy = coords[:, 1]
  cz = coords[:, 2]
  we1a = We1[:, :d]
  we1b = We1[:, d:2 * d]
  w3 = We1[:, 2 * d]
  be1r = be1.reshape(1, d)
  be2r = be2.reshape(1, d)
  bn1r = bn1.reshape(1, d)
  bn2r = bn2.reshape(1, d)
  bc1r = bc1.reshape(1, d)

  bn = 1000
  gridn = n // bn
  full = lambda shape: pl.BlockSpec(shape, lambda i: (0,) * len(shape))

  a_m, b_m, wp, bp = pl.pallas_call(
      _prep_body,
      grid=(gridn,),
      in_specs=[pl.BlockSpec((bn, d), lambda i: (i, 0)),
                full((d, d)), full((d, d)), full((1, d)),
                full((d, d)), full((d, d)), full((1, d)), full((1, d))],
      out_specs=[pl.BlockSpec((bn, d), lambda i: (i, 0)),
                 pl.BlockSpec((bn, d), lambda i: (i, 0)),
                 full((d, d)), full((1, d))],
      out_shape=[jax.ShapeDtypeStruct((n, d), jnp.float32),
                 jax.ShapeDtypeStruct((n, d), jnp.float32),
                 jax.ShapeDtypeStruct((d, d), jnp.float32),
                 jax.ShapeDtypeStruct((1, d), jnp.float32)],
  )(h, we1a, we1b, be1r, We2, Wc1, be2r, bc1r)

  mesh = plsc.VectorSubcoreMesh(core_axis_name="c", subcore_axis_name="s")
  dbl = lambda shape, dt: [pltpu.VMEM(shape, dt), pltpu.VMEM(shape, dt)]
  edge_sc = functools.partial(
      pl.kernel, mesh=mesh,
      out_type=[jax.ShapeDtypeStruct((e, d), jnp.float32),
                jax.ShapeDtypeStruct((e,), jnp.float32),
                jax.ShapeDtypeStruct((e,), jnp.float32),
                jax.ShapeDtypeStruct((e,), jnp.float32),
                jax.ShapeDtypeStruct((NC, n, d), jnp.float32)],
      scratch_types=(
          dbl((C,), jnp.int32) + dbl((C,), jnp.int32)
          + dbl((C, d), jnp.float32) + dbl((C, d), jnp.float32)
          + dbl((C,), jnp.float32) * 6
          + dbl((C, d), jnp.float32)
          + dbl((C,), jnp.float32) * 3
          + [pltpu.VMEM((d,), jnp.float32),
             pltpu.SMEM((C,), jnp.float32)]
          + [pltpu.VMEM_SHARED((n, d), jnp.float32)]
          + [pltpu.SemaphoreType.DMA] * 6
      ))(_edge_sc_body)
  r_h, relx, rely, relz, aggp = edge_sc(a_m, b_m, cx, cy, cz, row, col, w3)

  be = 2000
  cw = pl.pallas_call(
      _edge_mlp_body,
      grid=(e // be,),
      in_specs=[pl.BlockSpec((be, d), lambda i: (i, 0)),
                full((d, d)), full((1, d)), full((1, d))],
      out_specs=pl.BlockSpec((be, 1), lambda i: (i, 0)),
      out_shape=jax.ShapeDtypeStruct((e, 1), jnp.float32),
  )(r_h, wp, bp, Wc2)

  coord_sc = functools.partial(
      pl.kernel, mesh=mesh,
      out_type=[jax.ShapeDtypeStruct((NC * n,), jnp.float32)] * 4,
      scratch_types=(
          [pltpu.VMEM((kpts, 1, CS), jnp.int32)]
          + [pltpu.VMEM((kpts * CS,), jnp.float32)] * 7
          + [pltpu.VMEM((CS,), jnp.float32), pltpu.VMEM((ZR,), jnp.float32)]
          + [pltpu.VMEM_SHARED((n,), jnp.float32)] * 4
          + [pltpu.SemaphoreType.DMA]
      ))(_coord_sc_body)
  cupx, cupy, cupz, cdeg = coord_sc(cw.reshape(e), relx, rely, relz, row4)

  cup4 = lambda a: a.reshape(NC, gridn, 1, bn)
  cup_spec = pl.BlockSpec((NC, 1, 1, bn), lambda i: (0, i, 0, 0))
  hnew, cout = pl.pallas_call(
      _node_body,
      grid=(gridn,),
      in_specs=[pl.BlockSpec((bn, d), lambda i: (i, 0)),
                pl.BlockSpec((bn, d), lambda i: (i, 0)),
                pl.BlockSpec((bn, d), lambda i: (i, 0)),
                cup_spec, cup_spec, cup_spec, cup_spec,
                pl.BlockSpec((bn, 3), lambda i: (i, 0)),
                full((d, d)), full((1, d)), full((d, d)), full((d, d)),
                full((1, d)), full((d, d)), full((1, d))],
      out_specs=[pl.BlockSpec((bn, d), lambda i: (i, 0)),
                 pl.BlockSpec((bn, 3), lambda i: (i, 0))],
      out_shape=[jax.ShapeDtypeStruct((n, d), jnp.float32),
                 jax.ShapeDtypeStruct((n, 3), jnp.float32)],
  )(h, aggp[0], aggp[1], cup4(cupx), cup4(cupy), cup4(cupz), cup4(cdeg),
    coords, We2, be2r,
    Wn1[:, :d], Wn1[:, d:], bn1r, Wn2, bn2r)

  return hnew, cout


# speedup vs baseline: 1.0036x; 1.0036x over previous
import functools

import jax
import jax.numpy as jnp
from jax import lax
from jax.experimental import pallas as pl
from jax.experimental.pallas import tpu as pltpu
from jax.experimental.pallas import tpu_sc as plsc

NC = 2
NS = 16
NW = NC * NS
L = 16
C = 40
GS = (0, 16, 24)
CS = 80
ZR = 128


def _prep_body(h_ref, wa_ref, wb_ref, be1_ref, we2_ref, wc1_ref, be2_ref,
               bc1_ref, a_ref, b_ref, wp_ref, bp_ref):
  hb = h_ref[...]
  a_ref[...] = jnp.dot(hb, wa_ref[...].T, preferred_element_type=jnp.float32) + be1_ref[...]
  b_ref[...] = jnp.dot(hb, wb_ref[...].T, preferred_element_type=jnp.float32)
  wp_ref[...] = jnp.dot(we2_ref[...].T, wc1_ref[...].T,
                        preferred_element_type=jnp.float32)
  bp_ref[...] = jnp.dot(be2_ref[...], wc1_ref[...].T,
                        preferred_element_type=jnp.float32) + bc1_ref[...]


def _edge_mlp_body(r_ref, wp_ref, bp_ref, wc2_ref, cw_ref):
  t = jnp.maximum(
      jnp.dot(r_ref[...], wp_ref[...], preferred_element_type=jnp.float32)
      + bp_ref[...], 0.0)
  cw_ref[...] = jnp.sum(t * wc2_ref[...], axis=1, keepdims=True)


def _node_body(h_ref, agg0_ref, agg1_ref, cux_ref, cuy_ref, cuz_ref, cd_ref,
               coords_ref, we2_ref, be2_ref, wn1a_ref, wn1b_ref, bn1_ref,
               wn2_ref, bn2_ref, hnew_ref, cout_ref):
  bn = h_ref.shape[0]
  rsum = agg0_ref[...] + agg1_ref[...]
  deg = jnp.sum(cd_ref[...].reshape(NC, bn).T, axis=1, keepdims=True)
  aggm = jnp.dot(rsum, we2_ref[...].T, preferred_element_type=jnp.float32) \
      + deg * be2_ref[...]
  pre = (jnp.dot(h_ref[...], wn1a_ref[...].T, preferred_element_type=jnp.float32)
         + jnp.dot(aggm, wn1b_ref[...].T, preferred_element_type=jnp.float32)
         + bn1_ref[...])
  hnew_ref[...] = jnp.dot(jnp.maximum(pre, 0.0), wn2_ref[...].T,
                          preferred_element_type=jnp.float32) + bn2_ref[...]
  ux = jnp.sum(cux_ref[...].reshape(NC, bn).T, axis=1, keepdims=True)
  uy = jnp.sum(cuy_ref[...].reshape(NC, bn).T, axis=1, keepdims=True)
  uz = jnp.sum(cuz_ref[...].reshape(NC, bn).T, axis=1, keepdims=True)
  cout_ref[...] = coords_ref[...] + jnp.concatenate([ux, uy, uz], axis=1)


def _zero_vmem2(ref, nrows, ncols):
  z = jnp.zeros((L,), jnp.float32)
  def body(i, _):
    for j in range(ncols // L):
      ref[i, pl.ds(j * L, L)] = z
    return 0
  lax.fori_loop(0, nrows, body, 0)


def _tile_rows(n):
  rpt = (-(-n // NS) + 7) // 8 * 8
  last = n - (NS - 1) * rpt
  assert last > 0
  return rpt, last


def _fill_rows(zsrc, nz, dst, start, cnt):
  for t in range(0, cnt, nz):
    c = min(nz, cnt - t)
    pltpu.sync_copy(zsrc.at[pl.ds(0, c)], dst.at[pl.ds(start + t, c)])


def _per_tile_ranges(sid, n, fn):
  rpt, last = _tile_rows(n)
  @pl.when(sid < NS - 1)
  def _():
    fn(sid * rpt, rpt)
  @pl.when(sid == NS - 1)
  def _():
    fn((NS - 1) * rpt, last)


def _edge_sc_body(a_hbm, b_hbm, cx_hbm, cy_hbm, cz_hbm, row_hbm, col_hbm,
                  w3_hbm, rh_hbm, rx_hbm, ry_hbm, rz_hbm, aggp_hbm,
                  rowv0, rowv1, colv0, colv1, arows0, arows1, brows0, brows1,
                  xr0, xr1, yr0, yr1, zr0, zr1, xc0, xc1, yc0, yc1, zc0, zc1,
                  rbuf0, rbuf1, rxb0, rxb1, ryb0, ryb1,
                  rzb0, rzb1, rowsc0, rowsc1, w3v, sq_smem, aggsh,
                  isem0, isem1, gsem0, gsem1, wsem0, wsem1, ssem0, ssem1):
  n = aggsh.shape[0]
  d = arows0.shape[1]
  cid = lax.axis_index("c")
  sid = lax.axis_index("s")
  wid = sid * NC + cid
  e = row_hbm.shape[0]
  kpt = e // (NW * C)
  tile_e0 = wid * kpt * C

  rowv = (rowv0, rowv1); colv = (colv0, colv1)
  arows = (arows0, arows1); brows = (brows0, brows1)
  xr = (xr0, xr1); yr = (yr0, yr1); zr = (zr0, zr1)
  xc = (xc0, xc1); yc = (yc0, yc1); zc = (zc0, zc1)
  rbuf = (rbuf0, rbuf1)
  rxb = (rxb0, rxb1); ryb = (ryb0, ryb1); rzb = (rzb0, rzb1)
  rowsc = (rowsc0, rowsc1)
  isem = (isem0, isem1); gsem = (gsem0, gsem1); wsem = (wsem0, wsem1)
  ssem = (ssem0, ssem1)

  pltpu.sync_copy(w3_hbm, w3v)
  w3vecs = [w3v[pl.ds(j * L, L)] for j in range(d // L)]

  _zero_vmem2(rbuf0, C, d)
  _per_tile_ranges(sid, n, lambda s, c: _fill_rows(rbuf0, C, aggsh, s, c))
  plsc.subcore_barrier()

  def idescs(k, b):
    return [(row_hbm.at[pl.ds(tile_e0 + k * C, C)], rowv[b]),
            (col_hbm.at[pl.ds(tile_e0 + k * C, C)], colv[b])]

  def gdescs(b):
    return [(a_hbm.at[rowv[b]], arows[b]),
            (b_hbm.at[colv[b]], brows[b]),
            (cx_hbm.at[rowv[b]], xr[b]),
            (cy_hbm.at[rowv[b]], yr[b]),
            (cz_hbm.at[rowv[b]], zr[b]),
            (cx_hbm.at[colv[b]], xc[b]),
            (cy_hbm.at[colv[b]], yc[b]),
            (cz_hbm.at[colv[b]], zc[b])]

  def wdescs(k, b):
    base = tile_e0 + k * C
    return [(rbuf[b], rh_hbm.at[pl.ds(base, C)]),
            (rxb[b], rx_hbm.at[pl.ds(base, C)]),
            (ryb[b], ry_hbm.at[pl.ds(base, C)]),
            (rzb[b], rz_hbm.at[pl.ds(base, C)])]

  def sdesc(b):
    return (rbuf[b], aggsh.at[rowsc[b]])

  def process(k, b):
    @pl.when(k >= 2)
    def _():
      for src, dst in wdescs(k, b):
        pltpu.make_async_copy(src, dst, wsem[b]).wait()
      src, dst = sdesc(b)
      pltpu.make_async_copy(src, dst, ssem[b]).wait()
    for src, dst in gdescs(b):
      pltpu.make_async_copy(src, dst, gsem[b]).wait()
    @pl.when((k >= 1) & (k < kpt - 1))
    def _():
      for src, dst in idescs(k + 1, b ^ 1):
        pltpu.make_async_copy(src, dst, isem[b ^ 1]).wait()
    @pl.when(k < kpt - 1)
    def _():
      for src, dst in gdescs(b ^ 1):
        pltpu.async_copy(src, dst, gsem[b ^ 1])

    for g in GS:
      gs = pl.ds(g, L)
      relx = xr[b][gs] - xc[b][gs]
      rely = yr[b][gs] - yc[b][gs]
      relz = zr[b][gs] - zc[b][gs]
      rxb[b][gs] = relx
      ryb[b][gs] = rely
      rzb[b][gs] = relz
      sqv = relx * relx + rely * rely + relz * relz
      for lane in range(L):
        sq_smem[g + lane] = sqv[lane]

    def edge_body(e2, _):
      for u in range(2):
        ei = e2 * 2 + u
        sq = sq_smem[ei]
        for j in range(d // L):
          va = arows[b][ei, pl.ds(j * L, L)]
          vb = brows[b][ei, pl.ds(j * L, L)]
          rbuf[b][ei, pl.ds(j * L, L)] = jnp.maximum(
              va + vb + sq * w3vecs[j], 0.0)
      return 0
    lax.fori_loop(0, C // 2, edge_body, 0)

    for src, dst in wdescs(k, b):
      pltpu.async_copy(src, dst, wsem[b])
    for t in GS:
      rowsc[b][pl.ds(t, L)] = rowv[b][pl.ds(t, L)]
    src, dst = sdesc(b)
    pltpu.async_copy(src, dst, ssem[b], add=True)
    @pl.when(k < kpt - 2)
    def _():
      for src, dst in idescs(k + 2, b):
        pltpu.async_copy(src, dst, isem[b])

  for src, dst in idescs(0, 0):
    pltpu.sync_copy(src, dst)
  for src, dst in idescs(1, 1):
    pltpu.sync_copy(src, dst)
  for src, dst in gdescs(0):
    pltpu.async_copy(src, dst, gsem[0])

  def pair_body(p, _):
    for b in range(2):
      process(p * 2 + b, b)
    return 0
  assert True
  lax.fori_loop(0, kpt // 2, pair_body, 0)
  if kpt % 2:
    process(kpt - 1, (kpt - 1) % 2)

  for b in range(2):
    klast = kpt - 1 - ((kpt - 1 + b) % 2)
    for src, dst in wdescs(klast, b):
      pltpu.make_async_copy(src, dst, wsem[b]).wait()
    src, dst = sdesc(b)
    pltpu.make_async_copy(src, dst, ssem[b]).wait()

  plsc.subcore_barrier()
  _per_tile_ranges(
      sid, n,
      lambda s, c: pltpu.sync_copy(aggsh.at[pl.ds(s, c)],
                                   aggp_hbm.at[cid, pl.ds(s, c)]))


def _coord_sc_body(cw_hbm, rx_hbm, ry_hbm, rz_hbm, row4_hbm,
                   cupx_hbm, cupy_hbm, cupz_hbm, cdeg_hbm,
                   rowa, cwa, rxa, rya, rza, pxa, pya, pza, onesb, zb,
                   cshx, cshy, cshz, cshd, ssem):
  n = cshx.shape[0]
  cid = lax.axis_index("c")
  sid = lax.axis_index("s")
  wid = sid * NC + cid
  kpt = rowa.shape[0]
  ept = kpt * CS
  tile_e0 = wid * ept

  one = jnp.full((L,), 1.0, jnp.float32)
  for t in range(CS // L):
    onesb[pl.ds(t * L, L)] = one
  for t in range(ZR // L):
    zb[pl.ds(t * L, L)] = jnp.zeros((L,), jnp.float32)
  for sh in (cshx, cshy, cshz, cshd):
    _per_tile_ranges(sid, n, lambda s, c, sh=sh: _fill_rows(zb, ZR, sh, s, c))

  pltpu.sync_copy(row4_hbm.at[wid], rowa)
  pltpu.sync_copy(cw_hbm.at[pl.ds(tile_e0, ept)], cwa)
  pltpu.sync_copy(rx_hbm.at[pl.ds(tile_e0, ept)], rxa)
  pltpu.sync_copy(ry_hbm.at[pl.ds(tile_e0, ept)], rya)
  pltpu.sync_copy(rz_hbm.at[pl.ds(tile_e0, ept)], rza)

  def prod_body(i, _):
    s = pl.ds(i * L, L)
    w = cwa[s]
    pxa[s] = w * rxa[s]
    pya[s] = w * rya[s]
    pza[s] = w * rza[s]
    return 0
  lax.fori_loop(0, ept // L, prod_body, 0)
  plsc.subcore_barrier()

  def sdescs(k):
    es = pl.ds(k * CS, CS)
    return [(pxa.at[es], cshx.at[rowa.at[k, 0]]),
            (pya.at[es], cshy.at[rowa.at[k, 0]]),
            (pza.at[es], cshz.at[rowa.at[k, 0]]),
            (onesb, cshd.at[rowa.at[k, 0]])]

  def chunk_body(k, _):
    for src, dst in sdescs(k):
      pltpu.async_copy(src, dst, ssem, add=True)
    @pl.when(k >= 1)
    def _():
      for src, dst in sdescs(k - 1):
        pltpu.make_async_copy(src, dst, ssem).wait()
    return 0
  lax.fori_loop(0, kpt, chunk_body, 0)
  for src, dst in sdescs(kpt - 1):
    pltpu.make_async_copy(src, dst, ssem).wait()

  plsc.subcore_barrier()
  def _staged_out(sh, out, s, c):
    for t in range(0, c, ZR):
      cc = min(ZR, c - t)
      pltpu.sync_copy(sh.at[pl.ds(s + t, cc)], pxa.at[pl.ds(0, cc)])
      pltpu.sync_copy(pxa.at[pl.ds(0, cc)],
                      out.at[pl.ds(cid * n + s + t, cc)])
  for sh, out in ((cshx, cupx_hbm), (cshy, cupy_hbm), (cshz, cupz_hbm),
                  (cshd, cdeg_hbm)):
    _per_tile_ranges(
        sid, n,
        lambda s, c, sh=sh, out=out: _staged_out(sh, out, s, c))


def kernel(h, coords, edge_index, We1, be1, We2, be2, Wn1, bn1, Wn2, bn2,
           Wc1, bc1, Wc2):
  n, d = h.shape
  e = edge_index.shape[1]
  assert e % (NW * C) == 0 and e % (NW * CS) == 0 and d % L == 0
  kpts = e // (NW * CS)

  row = edge_index[0]
  col = edge_index[1]
  row4 = row.reshape(NW, kpts, 1, CS)
  cx = coords[:, 0]
  cy = coords[:, 1]
  cz = coords[:, 2]
  we1a = We1[:, :d]
  we1b = We1[:, d:2 * d]
  w3 = We1[:, 2 * d]
  be1r = be1.reshape(1, d)
  be2r = be2.reshape(1, d)
  bn1r = bn1.reshape(1, d)
  bn2r = bn2.reshape(1, d)
  bc1r = bc1.reshape(1, d)

  bn = 1000
  gridn = n // bn
  full = lambda shape: pl.BlockSpec(shape, lambda i: (0,) * len(shape))

  a_m, b_m, wp, bp = pl.pallas_call(
      _prep_body,
      grid=(gridn,),
      in_specs=[pl.BlockSpec((bn, d), lambda i: (i, 0)),
                full((d, d)), full((d, d)), full((1, d)),
                full((d, d)), full((d, d)), full((1, d)), full((1, d))],
      out_specs=[pl.BlockSpec((bn, d), lambda i: (i, 0)),
                 pl.BlockSpec((bn, d), lambda i: (i, 0)),
                 full((d, d)), full((1, d))],
      out_shape=[jax.ShapeDtypeStruct((n, d), jnp.float32),
                 jax.ShapeDtypeStruct((n, d), jnp.float32),
                 jax.ShapeDtypeStruct((d, d), jnp.float32),
                 jax.ShapeDtypeStruct((1, d), jnp.float32)],
  )(h, we1a, we1b, be1r, We2, Wc1, be2r, bc1r)

  mesh = plsc.VectorSubcoreMesh(core_axis_name="c", subcore_axis_name="s")
  dbl = lambda shape, dt: [pltpu.VMEM(shape, dt), pltpu.VMEM(shape, dt)]
  edge_sc = functools.partial(
      pl.kernel, mesh=mesh,
      out_type=[jax.ShapeDtypeStruct((e, d), jnp.float32),
                jax.ShapeDtypeStruct((e,), jnp.float32),
                jax.ShapeDtypeStruct((e,), jnp.float32),
                jax.ShapeDtypeStruct((e,), jnp.float32),
                jax.ShapeDtypeStruct((NC, n, d), jnp.float32)],
      scratch_types=(
          dbl((C,), jnp.int32) + dbl((C,), jnp.int32)
          + dbl((C, d), jnp.float32) + dbl((C, d), jnp.float32)
          + dbl((C,), jnp.float32) * 6
          + dbl((C, d), jnp.float32)
          + dbl((C,), jnp.float32) * 3
          + dbl((C,), jnp.int32)
          + [pltpu.VMEM((d,), jnp.float32),
             pltpu.SMEM((C,), jnp.float32)]
          + [pltpu.VMEM_SHARED((n, d), jnp.float32)]
          + [pltpu.SemaphoreType.DMA] * 8
      ))(_edge_sc_body)
  r_h, relx, rely, relz, aggp = edge_sc(a_m, b_m, cx, cy, cz, row, col, w3)

  be = 2000
  cw = pl.pallas_call(
      _edge_mlp_body,
      grid=(e // be,),
      in_specs=[pl.BlockSpec((be, d), lambda i: (i, 0)),
                full((d, d)), full((1, d)), full((1, d))],
      out_specs=pl.BlockSpec((be, 1), lambda i: (i, 0)),
      out_shape=jax.ShapeDtypeStruct((e, 1), jnp.float32),
  )(r_h, wp, bp, Wc2)

  coord_sc = functools.partial(
      pl.kernel, mesh=mesh,
      out_type=[jax.ShapeDtypeStruct((NC * n,), jnp.float32)] * 4,
      scratch_types=(
          [pltpu.VMEM((kpts, 1, CS), jnp.int32)]
          + [pltpu.VMEM((kpts * CS,), jnp.float32)] * 7
          + [pltpu.VMEM((CS,), jnp.float32), pltpu.VMEM((ZR,), jnp.float32)]
          + [pltpu.VMEM_SHARED((n,), jnp.float32)] * 4
          + [pltpu.SemaphoreType.DMA]
      ))(_coord_sc_body)
  cupx, cupy, cupz, cdeg = coord_sc(cw.reshape(e), relx, rely, relz, row4)

  cup4 = lambda a: a.reshape(NC, gridn, 1, bn)
  cup_spec = pl.BlockSpec((NC, 1, 1, bn), lambda i: (0, i, 0, 0))
  hnew, cout = pl.pallas_call(
      _node_body,
      grid=(gridn,),
      in_specs=[pl.BlockSpec((bn, d), lambda i: (i, 0)),
                pl.BlockSpec((bn, d), lambda i: (i, 0)),
                pl.BlockSpec((bn, d), lambda i: (i, 0)),
                cup_spec, cup_spec, cup_spec, cup_spec,
                pl.BlockSpec((bn, 3), lambda i: (i, 0)),
                full((d, d)), full((1, d)), full((d, d)), full((d, d)),
                full((1, d)), full((d, d)), full((1, d))],
      out_specs=[pl.BlockSpec((bn, d), lambda i: (i, 0)),
                 pl.BlockSpec((bn, 3), lambda i: (i, 0))],
      out_shape=[jax.ShapeDtypeStruct((n, d), jnp.float32),
                 jax.ShapeDtypeStruct((n, 3), jnp.float32)],
  )(h, aggp[0], aggp[1], cup4(cupx), cup4(cupy), cup4(cupz), cup4(cdeg),
    coords, We2, be2r,
    Wn1[:, :d], Wn1[:, d:], bn1r, Wn2, bn2r)

  return hnew, cout


# speedup vs baseline: 1.0224x; 1.0187x over previous
import functools

import jax
import jax.numpy as jnp
from jax import lax
from jax.experimental import pallas as pl
from jax.experimental.pallas import tpu as pltpu
from jax.experimental.pallas import tpu_sc as plsc

NC = 2
NS = 16
NW = NC * NS
L = 16
C = 40
GS = (0, 16, 24)
CS = 80
ZR = 128


def _prep_body(h_ref, wa_ref, wb_ref, be1_ref, we2_ref, wc1_ref, be2_ref,
               bc1_ref, a_ref, b_ref, wp_ref, bp_ref):
  hb = h_ref[...]
  a_ref[...] = jnp.dot(hb, wa_ref[...].T, preferred_element_type=jnp.float32) + be1_ref[...]
  b_ref[...] = jnp.dot(hb, wb_ref[...].T, preferred_element_type=jnp.float32)
  wp_ref[...] = jnp.dot(we2_ref[...].T, wc1_ref[...].T,
                        preferred_element_type=jnp.float32)
  bp_ref[...] = jnp.dot(be2_ref[...], wc1_ref[...].T,
                        preferred_element_type=jnp.float32) + bc1_ref[...]


def _edge_mlp_body(r_ref, wp_ref, bp_ref, wc2_ref, cw_ref):
  t = jnp.maximum(
      jnp.dot(r_ref[...], wp_ref[...], preferred_element_type=jnp.float32)
      + bp_ref[...], 0.0)
  cw_ref[...] = jnp.sum(t * wc2_ref[...], axis=1, keepdims=True)


def _node_body(h_ref, agg0_ref, agg1_ref, agg2_ref, agg3_ref,
               cux_ref, cuy_ref, cuz_ref, cd_ref,
               coords_ref, we2_ref, be2_ref, wn1a_ref, wn1b_ref, bn1_ref,
               wn2_ref, bn2_ref, hnew_ref, cout_ref):
  bn = h_ref.shape[0]
  rsum = (agg0_ref[...] + agg1_ref[...]) + (agg2_ref[...] + agg3_ref[...])
  deg = jnp.sum(cd_ref[...].reshape(NC, bn).T, axis=1, keepdims=True)
  aggm = jnp.dot(rsum, we2_ref[...].T, preferred_element_type=jnp.float32) \
      + deg * be2_ref[...]
  pre = (jnp.dot(h_ref[...], wn1a_ref[...].T, preferred_element_type=jnp.float32)
         + jnp.dot(aggm, wn1b_ref[...].T, preferred_element_type=jnp.float32)
         + bn1_ref[...])
  hnew_ref[...] = jnp.dot(jnp.maximum(pre, 0.0), wn2_ref[...].T,
                          preferred_element_type=jnp.float32) + bn2_ref[...]
  ux = jnp.sum(cux_ref[...].reshape(NC, bn).T, axis=1, keepdims=True)
  uy = jnp.sum(cuy_ref[...].reshape(NC, bn).T, axis=1, keepdims=True)
  uz = jnp.sum(cuz_ref[...].reshape(NC, bn).T, axis=1, keepdims=True)
  cout_ref[...] = coords_ref[...] + jnp.concatenate([ux, uy, uz], axis=1)


def _zero_vmem2(ref, nrows, ncols):
  z = jnp.zeros((L,), jnp.float32)
  def body(i, _):
    for j in range(ncols // L):
      ref[i, pl.ds(j * L, L)] = z
    return 0
  lax.fori_loop(0, nrows, body, 0)


def _tile_rows(n):
  rpt = (-(-n // NS) + 7) // 8 * 8
  last = n - (NS - 1) * rpt
  assert last > 0
  return rpt, last


def _fill_rows(zsrc, nz, dst, start, cnt):
  for t in range(0, cnt, nz):
    c = min(nz, cnt - t)
    pltpu.sync_copy(zsrc.at[pl.ds(0, c)], dst.at[pl.ds(start + t, c)])


def _per_tile_ranges(sid, n, fn):
  rpt, last = _tile_rows(n)
  @pl.when(sid < NS - 1)
  def _():
    fn(sid * rpt, rpt)
  @pl.when(sid == NS - 1)
  def _():
    fn((NS - 1) * rpt, last)


def _edge_sc_body(a_hbm, b_hbm, cx_hbm, cy_hbm, cz_hbm, row_hbm, col_hbm,
                  w3_hbm, rh_hbm, rx_hbm, ry_hbm, rz_hbm, aggp_hbm,
                  rowv0, rowv1, colv0, colv1, arows0, arows1, brows0, brows1,
                  xr0, xr1, yr0, yr1, zr0, zr1, xc0, xc1, yc0, yc1, zc0, zc1,
                  rbuf0, rbuf1, rxb0, rxb1, ryb0, ryb1,
                  rzb0, rzb1, rowsc0, rowsc1, w3v, sq_smem, aggsh,
                  isem0, isem1, gsem0, gsem1, wsem0, wsem1, ssem0, ssem1):
  n = aggsh.shape[0]
  d = arows0.shape[1]
  cid = lax.axis_index("c")
  sid = lax.axis_index("s")
  wid = sid * NC + cid
  e = row_hbm.shape[0]
  kpt = e // (NW * C)
  tile_e0 = wid * kpt * C

  rowv = (rowv0, rowv1); colv = (colv0, colv1)
  arows = (arows0, arows1); brows = (brows0, brows1)
  xr = (xr0, xr1); yr = (yr0, yr1); zr = (zr0, zr1)
  xc = (xc0, xc1); yc = (yc0, yc1); zc = (zc0, zc1)
  rbuf = (rbuf0, rbuf1)
  rxb = (rxb0, rxb1); ryb = (ryb0, ryb1); rzb = (rzb0, rzb1)
  rowsc = (rowsc0, rowsc1)
  isem = (isem0, isem1); gsem = (gsem0, gsem1); wsem = (wsem0, wsem1)
  ssem = (ssem0, ssem1)

  pltpu.sync_copy(w3_hbm, w3v)
  w3vecs = [w3v[pl.ds(j * L, L)] for j in range(d // L)]

  _zero_vmem2(rbuf0, C, d)
  _per_tile_ranges(sid, n, lambda s, c: _fill_rows(rbuf0, C, aggsh, s, c))
  plsc.subcore_barrier()

  def idescs(k, b):
    return [(row_hbm.at[pl.ds(tile_e0 + k * C, C)], rowv[b]),
            (col_hbm.at[pl.ds(tile_e0 + k * C, C)], colv[b])]

  def gdescs(b):
    return [(a_hbm.at[rowv[b]], arows[b]),
            (b_hbm.at[colv[b]], brows[b]),
            (cx_hbm.at[rowv[b]], xr[b]),
            (cy_hbm.at[rowv[b]], yr[b]),
            (cz_hbm.at[rowv[b]], zr[b]),
            (cx_hbm.at[colv[b]], xc[b]),
            (cy_hbm.at[colv[b]], yc[b]),
            (cz_hbm.at[colv[b]], zc[b])]

  def wdescs(k, b):
    base = tile_e0 + k * C
    return [(rbuf[b], rh_hbm.at[pl.ds(base, C)]),
            (rxb[b], rx_hbm.at[pl.ds(base, C)]),
            (ryb[b], ry_hbm.at[pl.ds(base, C)]),
            (rzb[b], rz_hbm.at[pl.ds(base, C)])]

  def sdesc(b):
    return (rbuf[b], aggsh.at[rowsc[b]])

  def process(k, b):
    @pl.when(k >= 2)
    def _():
      for src, dst in wdescs(k, b):
        pltpu.make_async_copy(src, dst, wsem[b]).wait()
      src, dst = sdesc(b)
      pltpu.make_async_copy(src, dst, ssem[b]).wait()
    for src, dst in gdescs(b):
      pltpu.make_async_copy(src, dst, gsem[b]).wait()
    @pl.when((k >= 1) & (k < kpt - 1))
    def _():
      for src, dst in idescs(k + 1, b ^ 1):
        pltpu.make_async_copy(src, dst, isem[b ^ 1]).wait()
    @pl.when(k < kpt - 1)
    def _():
      for src, dst in gdescs(b ^ 1):
        pltpu.async_copy(src, dst, gsem[b ^ 1])

    for g in GS:
      gs = pl.ds(g, L)
      relx = xr[b][gs] - xc[b][gs]
      rely = yr[b][gs] - yc[b][gs]
      relz = zr[b][gs] - zc[b][gs]
      rxb[b][gs] = relx
      ryb[b][gs] = rely
      rzb[b][gs] = relz
      sqv = relx * relx + rely * rely + relz * relz
      for lane in range(L):
        sq_smem[g + lane] = sqv[lane]

    def edge_body(e2, _):
      for u in range(2):
        ei = e2 * 2 + u
        sq = sq_smem[ei]
        for j in range(d // L):
          va = arows[b][ei, pl.ds(j * L, L)]
          vb = brows[b][ei, pl.ds(j * L, L)]
          rbuf[b][ei, pl.ds(j * L, L)] = jnp.maximum(
              va + vb + sq * w3vecs[j], 0.0)
      return 0
    lax.fori_loop(0, C // 2, edge_body, 0)

    for src, dst in wdescs(k, b):
      pltpu.async_copy(src, dst, wsem[b])
    for t in GS:
      rowsc[b][pl.ds(t, L)] = rowv[b][pl.ds(t, L)]
    src, dst = sdesc(b)
    pltpu.async_copy(src, dst, ssem[b], add=True)
    @pl.when(k < kpt - 2)
    def _():
      for src, dst in idescs(k + 2, b):
        pltpu.async_copy(src, dst, isem[b])

  for src, dst in idescs(0, 0):
    pltpu.sync_copy(src, dst)
  for src, dst in idescs(1, 1):
    pltpu.sync_copy(src, dst)
  for src, dst in gdescs(0):
    pltpu.async_copy(src, dst, gsem[0])

  def pair_body(p, _):
    for b in range(2):
      process(p * 2 + b, b)
    return 0
  assert True
  lax.fori_loop(0, kpt // 2, pair_body, 0)
  if kpt % 2:
    process(kpt - 1, (kpt - 1) % 2)

  for b in range(2):
    klast = kpt - 1 - ((kpt - 1 + b) % 2)
    for src, dst in wdescs(klast, b):
      pltpu.make_async_copy(src, dst, wsem[b]).wait()
    src, dst = sdesc(b)
    pltpu.make_async_copy(src, dst, ssem[b]).wait()

  plsc.subcore_barrier()
  _per_tile_ranges(
      sid, n,
      lambda s, c: pltpu.sync_copy(aggsh.at[pl.ds(s, c)],
                                   aggp_hbm.at[cid, pl.ds(s, c)]))


def _coord_sc_body(cw_hbm, rx_hbm, ry_hbm, rz_hbm, row4_hbm,
                   cupx_hbm, cupy_hbm, cupz_hbm, cdeg_hbm,
                   rowa, cwa, rxa, rya, rza, pxa, pya, pza, onesb, zb,
                   cshx, cshy, cshz, cshd, ssem):
  n = cshx.shape[0]
  cid = lax.axis_index("c")
  sid = lax.axis_index("s")
  wid = sid * NC + cid
  kpt = rowa.shape[0]
  ept = kpt * CS
  tile_e0 = wid * ept

  one = jnp.full((L,), 1.0, jnp.float32)
  for t in range(CS // L):
    onesb[pl.ds(t * L, L)] = one
  for t in range(ZR // L):
    zb[pl.ds(t * L, L)] = jnp.zeros((L,), jnp.float32)
  for sh in (cshx, cshy, cshz, cshd):
    _per_tile_ranges(sid, n, lambda s, c, sh=sh: _fill_rows(zb, ZR, sh, s, c))

  pltpu.sync_copy(row4_hbm.at[wid], rowa)
  pltpu.sync_copy(cw_hbm.at[pl.ds(tile_e0, ept)], cwa)
  pltpu.sync_copy(rx_hbm.at[pl.ds(tile_e0, ept)], rxa)
  pltpu.sync_copy(ry_hbm.at[pl.ds(tile_e0, ept)], rya)
  pltpu.sync_copy(rz_hbm.at[pl.ds(tile_e0, ept)], rza)

  def prod_body(i, _):
    s = pl.ds(i * L, L)
    w = cwa[s]
    pxa[s] = w * rxa[s]
    pya[s] = w * rya[s]
    pza[s] = w * rza[s]
    return 0
  lax.fori_loop(0, ept // L, prod_body, 0)
  plsc.subcore_barrier()

  def sdescs(k):
    es = pl.ds(k * CS, CS)
    return [(pxa.at[es], cshx.at[rowa.at[k, 0]]),
            (pya.at[es], cshy.at[rowa.at[k, 0]]),
            (pza.at[es], cshz.at[rowa.at[k, 0]]),
            (onesb, cshd.at[rowa.at[k, 0]])]

  def chunk_body(k, _):
    for src, dst in sdescs(k):
      pltpu.async_copy(src, dst, ssem, add=True)
    @pl.when(k >= 1)
    def _():
      for src, dst in sdescs(k - 1):
        pltpu.make_async_copy(src, dst, ssem).wait()
    return 0
  lax.fori_loop(0, kpt, chunk_body, 0)
  for src, dst in sdescs(kpt - 1):
    pltpu.make_async_copy(src, dst, ssem).wait()

  plsc.subcore_barrier()
  def _staged_out(sh, out, s, c):
    for t in range(0, c, ZR):
      cc = min(ZR, c - t)
      pltpu.sync_copy(sh.at[pl.ds(s + t, cc)], pxa.at[pl.ds(0, cc)])
      pltpu.sync_copy(pxa.at[pl.ds(0, cc)],
                      out.at[pl.ds(cid * n + s + t, cc)])
  for sh, out in ((cshx, cupx_hbm), (cshy, cupy_hbm), (cshz, cupz_hbm),
                  (cshd, cdeg_hbm)):
    _per_tile_ranges(
        sid, n,
        lambda s, c, sh=sh, out=out: _staged_out(sh, out, s, c))


def kernel(h, coords, edge_index, We1, be1, We2, be2, Wn1, bn1, Wn2, bn2,
           Wc1, bc1, Wc2):
  n, d = h.shape
  e = edge_index.shape[1]
  assert e % (NW * C) == 0 and e % (NW * CS) == 0 and d % L == 0
  kpts = e // (NW * CS)

  row = edge_index[0]
  col = edge_index[1]
  row4 = row.reshape(NW, kpts, 1, CS)
  cx = coords[:, 0]
  cy = coords[:, 1]
  cz = coords[:, 2]
  we1a = We1[:, :d]
  we1b = We1[:, d:2 * d]
  w3 = We1[:, 2 * d]
  be1r = be1.reshape(1, d)
  be2r = be2.reshape(1, d)
  bn1r = bn1.reshape(1, d)
  bn2r = bn2.reshape(1, d)
  bc1r = bc1.reshape(1, d)

  bn = 1000
  gridn = n // bn
  full = lambda shape: pl.BlockSpec(shape, lambda i: (0,) * len(shape))

  a_m, b_m, wp, bp = pl.pallas_call(
      _prep_body,
      grid=(gridn,),
      in_specs=[pl.BlockSpec((bn, d), lambda i: (i, 0)),
                full((d, d)), full((d, d)), full((1, d)),
                full((d, d)), full((d, d)), full((1, d)), full((1, d))],
      out_specs=[pl.BlockSpec((bn, d), lambda i: (i, 0)),
                 pl.BlockSpec((bn, d), lambda i: (i, 0)),
                 full((d, d)), full((1, d))],
      out_shape=[jax.ShapeDtypeStruct((n, d), jnp.float32),
                 jax.ShapeDtypeStruct((n, d), jnp.float32),
                 jax.ShapeDtypeStruct((d, d), jnp.float32),
                 jax.ShapeDtypeStruct((1, d), jnp.float32)],
  )(h, we1a, we1b, be1r, We2, Wc1, be2r, bc1r)

  mesh = plsc.VectorSubcoreMesh(core_axis_name="c", subcore_axis_name="s")
  dbl = lambda shape, dt: [pltpu.VMEM(shape, dt), pltpu.VMEM(shape, dt)]
  def make_edge_sc(esz):
    return functools.partial(
        pl.kernel, mesh=mesh,
        out_type=[jax.ShapeDtypeStruct((esz, d), jnp.float32),
                  jax.ShapeDtypeStruct((esz,), jnp.float32),
                  jax.ShapeDtypeStruct((esz,), jnp.float32),
                  jax.ShapeDtypeStruct((esz,), jnp.float32),
                  jax.ShapeDtypeStruct((NC, n, d), jnp.float32)],
        scratch_types=(
            dbl((C,), jnp.int32) + dbl((C,), jnp.int32)
            + dbl((C, d), jnp.float32) + dbl((C, d), jnp.float32)
            + dbl((C,), jnp.float32) * 6
            + dbl((C, d), jnp.float32)
            + dbl((C,), jnp.float32) * 3
            + dbl((C,), jnp.int32)
            + [pltpu.VMEM((d,), jnp.float32),
               pltpu.SMEM((C,), jnp.float32)]
            + [pltpu.VMEM_SHARED((n, d), jnp.float32)]
            + [pltpu.SemaphoreType.DMA] * 8
        ))(_edge_sc_body)

  e2 = e // 2
  assert e2 % (NW * C) == 0
  edge_sc = make_edge_sc(e2)
  r_1, rx1, ry1, rz1, aggp1 = edge_sc(a_m, b_m, cx, cy, cz,
                                      row[:e2], col[:e2], w3)
  r_2, rx2, ry2, rz2, aggp2 = edge_sc(a_m, b_m, cx, cy, cz,
                                      row[e2:], col[e2:], w3)

  be = 2000
  def edge_mlp(r_h):
    return pl.pallas_call(
        _edge_mlp_body,
        grid=(e2 // be,),
        in_specs=[pl.BlockSpec((be, d), lambda i: (i, 0)),
                  full((d, d)), full((1, d)), full((1, d))],
        out_specs=pl.BlockSpec((be, 1), lambda i: (i, 0)),
        out_shape=jax.ShapeDtypeStruct((e2, 1), jnp.float32),
    )(r_h, wp, bp, Wc2)
  cw = jnp.concatenate([edge_mlp(r_1), edge_mlp(r_2)], axis=0)
  relx = jnp.concatenate([rx1, rx2])
  rely = jnp.concatenate([ry1, ry2])
  relz = jnp.concatenate([rz1, rz2])

  coord_sc = functools.partial(
      pl.kernel, mesh=mesh,
      out_type=[jax.ShapeDtypeStruct((NC * n,), jnp.float32)] * 4,
      scratch_types=(
          [pltpu.VMEM((kpts, 1, CS), jnp.int32)]
          + [pltpu.VMEM((kpts * CS,), jnp.float32)] * 7
          + [pltpu.VMEM((CS,), jnp.float32), pltpu.VMEM((ZR,), jnp.float32)]
          + [pltpu.VMEM_SHARED((n,), jnp.float32)] * 4
          + [pltpu.SemaphoreType.DMA]
      ))(_coord_sc_body)
  cupx, cupy, cupz, cdeg = coord_sc(cw.reshape(e), relx, rely, relz, row4)

  cup4 = lambda a: a.reshape(NC, gridn, 1, bn)
  cup_spec = pl.BlockSpec((NC, 1, 1, bn), lambda i: (0, i, 0, 0))
  hnew, cout = pl.pallas_call(
      _node_body,
      grid=(gridn,),
      in_specs=[pl.BlockSpec((bn, d), lambda i: (i, 0)),
                pl.BlockSpec((bn, d), lambda i: (i, 0)),
                pl.BlockSpec((bn, d), lambda i: (i, 0)),
                pl.BlockSpec((bn, d), lambda i: (i, 0)),
                pl.BlockSpec((bn, d), lambda i: (i, 0)),
                cup_spec, cup_spec, cup_spec, cup_spec,
                pl.BlockSpec((bn, 3), lambda i: (i, 0)),
                full((d, d)), full((1, d)), full((d, d)), full((d, d)),
                full((1, d)), full((d, d)), full((1, d))],
      out_specs=[pl.BlockSpec((bn, d), lambda i: (i, 0)),
                 pl.BlockSpec((bn, 3), lambda i: (i, 0))],
      out_shape=[jax.ShapeDtypeStruct((n, d), jnp.float32),
                 jax.ShapeDtypeStruct((n, 3), jnp.float32)],
  )(h, aggp1[0], aggp1[1], aggp2[0], aggp2[1],
    cup4(cupx), cup4(cupy), cup4(cupz), cup4(cdeg),
    coords, We2, be2r,
    Wn1[:, :d], Wn1[:, d:], bn1r, Wn2, bn2r)

  return hnew, cout


# speedup vs baseline: 1.1703x; 1.1447x over previous
import functools

import jax
import jax.numpy as jnp
from jax import lax
from jax.experimental import pallas as pl
from jax.experimental.pallas import tpu as pltpu
from jax.experimental.pallas import tpu_sc as plsc

NC = 2
NS = 16
NW = NC * NS
L = 16
C = 40
GS = (0, 16, 24)
CS = 80
ZR = 128


def _prep_body(h_ref, wa_ref, wb_ref, be1_ref, we2_ref, wc1_ref, be2_ref,
               bc1_ref, a_ref, b_ref, wp_ref, bp_ref):
  hb = h_ref[...]
  a_ref[...] = jnp.dot(hb, wa_ref[...].T, preferred_element_type=jnp.float32) + be1_ref[...]
  b_ref[...] = jnp.dot(hb, wb_ref[...].T, preferred_element_type=jnp.float32)
  wp_ref[...] = jnp.dot(we2_ref[...].T, wc1_ref[...].T,
                        preferred_element_type=jnp.float32)
  bp_ref[...] = jnp.dot(be2_ref[...], wc1_ref[...].T,
                        preferred_element_type=jnp.float32) + bc1_ref[...]


def _edge_mlp_body(r_ref, wp_ref, bp_ref, wc2_ref, cw_ref):
  t = jnp.maximum(
      jnp.dot(r_ref[...], wp_ref[...], preferred_element_type=jnp.float32)
      + bp_ref[...], 0.0)
  cw = jnp.sum(t * wc2_ref[...], axis=1, keepdims=True)
  cw_ref[...] = cw.T.reshape(cw_ref.shape)


def _node_body(h_ref, agg0_ref, agg1_ref, agg2_ref, agg3_ref,
               cux_ref, cuy_ref, cuz_ref, cd_ref,
               coords_ref, we2_ref, be2_ref, wn1a_ref, wn1b_ref, bn1_ref,
               wn2_ref, bn2_ref, hnew_ref, cout_ref):
  bn = h_ref.shape[0]
  rsum = (agg0_ref[...] + agg1_ref[...]) + (agg2_ref[...] + agg3_ref[...])
  deg = jnp.sum(cd_ref[...].reshape(NC, bn).T, axis=1, keepdims=True)
  aggm = jnp.dot(rsum, we2_ref[...].T, preferred_element_type=jnp.float32) \
      + deg * be2_ref[...]
  pre = (jnp.dot(h_ref[...], wn1a_ref[...].T, preferred_element_type=jnp.float32)
         + jnp.dot(aggm, wn1b_ref[...].T, preferred_element_type=jnp.float32)
         + bn1_ref[...])
  hnew_ref[...] = jnp.dot(jnp.maximum(pre, 0.0), wn2_ref[...].T,
                          preferred_element_type=jnp.float32) + bn2_ref[...]
  ux = jnp.sum(cux_ref[...].reshape(NC, bn).T, axis=1, keepdims=True)
  uy = jnp.sum(cuy_ref[...].reshape(NC, bn).T, axis=1, keepdims=True)
  uz = jnp.sum(cuz_ref[...].reshape(NC, bn).T, axis=1, keepdims=True)
  cout_ref[...] = coords_ref[...] + jnp.concatenate([ux, uy, uz], axis=1)


def _zero_vmem2(ref, nrows, ncols):
  z = jnp.zeros((L,), jnp.float32)
  def body(i, _):
    for j in range(ncols // L):
      ref[i, pl.ds(j * L, L)] = z
    return 0
  lax.fori_loop(0, nrows, body, 0)


def _tile_rows(n):
  rpt = (-(-n // NS) + 7) // 8 * 8
  last = n - (NS - 1) * rpt
  assert last > 0
  return rpt, last


def _fill_rows(zsrc, nz, dst, start, cnt):
  for t in range(0, cnt, nz):
    c = min(nz, cnt - t)
    pltpu.sync_copy(zsrc.at[pl.ds(0, c)], dst.at[pl.ds(start + t, c)])


def _per_tile_ranges(sid, n, fn):
  rpt, last = _tile_rows(n)
  @pl.when(sid < NS - 1)
  def _():
    fn(sid * rpt, rpt)
  @pl.when(sid == NS - 1)
  def _():
    fn((NS - 1) * rpt, last)


def _edge_sc_body(a_hbm, b_hbm, cx_hbm, cy_hbm, cz_hbm, row_hbm, col_hbm,
                  w3_hbm, rh_hbm, rx_hbm, ry_hbm, rz_hbm, aggp_hbm,
                  rowv0, rowv1, colv0, colv1, arows0, arows1, brows0, brows1,
                  xr0, xr1, yr0, yr1, zr0, zr1, xc0, xc1, yc0, yc1, zc0, zc1,
                  rbuf0, rbuf1, rxb0, rxb1, ryb0, ryb1,
                  rzb0, rzb1, rowsc0, rowsc1, w3v, sq_smem, aggsh,
                  isem0, isem1, gsem0, gsem1, wsem0, wsem1, ssem0, ssem1):
  n = aggsh.shape[0]
  d = arows0.shape[1]
  cid = lax.axis_index("c")
  sid = lax.axis_index("s")
  wid = sid * NC + cid
  e = row_hbm.shape[0]
  kpt = e // (NW * C)
  tile_e0 = wid * kpt * C

  rowv = (rowv0, rowv1); colv = (colv0, colv1)
  arows = (arows0, arows1); brows = (brows0, brows1)
  xr = (xr0, xr1); yr = (yr0, yr1); zr = (zr0, zr1)
  xc = (xc0, xc1); yc = (yc0, yc1); zc = (zc0, zc1)
  rbuf = (rbuf0, rbuf1)
  rxb = (rxb0, rxb1); ryb = (ryb0, ryb1); rzb = (rzb0, rzb1)
  rowsc = (rowsc0, rowsc1)
  isem = (isem0, isem1); gsem = (gsem0, gsem1); wsem = (wsem0, wsem1)
  ssem = (ssem0, ssem1)

  pltpu.sync_copy(w3_hbm, w3v)
  w3vecs = [w3v[pl.ds(j * L, L)] for j in range(d // L)]

  _zero_vmem2(rbuf0, C, d)
  _per_tile_ranges(sid, n, lambda s, c: _fill_rows(rbuf0, C, aggsh, s, c))
  plsc.subcore_barrier()

  def idescs(k, b):
    return [(row_hbm.at[pl.ds(tile_e0 + k * C, C)], rowv[b]),
            (col_hbm.at[pl.ds(tile_e0 + k * C, C)], colv[b])]

  def gdescs(b):
    return [(a_hbm.at[rowv[b]], arows[b]),
            (b_hbm.at[colv[b]], brows[b]),
            (cx_hbm.at[rowv[b]], xr[b]),
            (cy_hbm.at[rowv[b]], yr[b]),
            (cz_hbm.at[rowv[b]], zr[b]),
            (cx_hbm.at[colv[b]], xc[b]),
            (cy_hbm.at[colv[b]], yc[b]),
            (cz_hbm.at[colv[b]], zc[b])]

  def wdescs(k, b):
    base = tile_e0 + k * C
    return [(rbuf[b], rh_hbm.at[pl.ds(base, C)]),
            (rxb[b], rx_hbm.at[pl.ds(base, C)]),
            (ryb[b], ry_hbm.at[pl.ds(base, C)]),
            (rzb[b], rz_hbm.at[pl.ds(base, C)])]

  def sdesc(b):
    return (rbuf[b], aggsh.at[rowsc[b]])

  def process(k, b):
    @pl.when(k >= 2)
    def _():
      for src, dst in wdescs(k, b):
        pltpu.make_async_copy(src, dst, wsem[b]).wait()
      src, dst = sdesc(b)
      pltpu.make_async_copy(src, dst, ssem[b]).wait()
    for src, dst in gdescs(b):
      pltpu.make_async_copy(src, dst, gsem[b]).wait()
    @pl.when((k >= 1) & (k < kpt - 1))
    def _():
      for src, dst in idescs(k + 1, b ^ 1):
        pltpu.make_async_copy(src, dst, isem[b ^ 1]).wait()
    @pl.when(k < kpt - 1)
    def _():
      for src, dst in gdescs(b ^ 1):
        pltpu.async_copy(src, dst, gsem[b ^ 1])

    for g in GS:
      gs = pl.ds(g, L)
      relx = xr[b][gs] - xc[b][gs]
      rely = yr[b][gs] - yc[b][gs]
      relz = zr[b][gs] - zc[b][gs]
      rxb[b][gs] = relx
      ryb[b][gs] = rely
      rzb[b][gs] = relz
      sqv = relx * relx + rely * rely + relz * relz
      for lane in range(L):
        sq_smem[g + lane] = sqv[lane]

    def edge_body(e2, _):
      for u in range(2):
        ei = e2 * 2 + u
        sq = sq_smem[ei]
        for j in range(d // L):
          va = arows[b][ei, pl.ds(j * L, L)]
          vb = brows[b][ei, pl.ds(j * L, L)]
          rbuf[b][ei, pl.ds(j * L, L)] = jnp.maximum(
              va + vb + sq * w3vecs[j], 0.0)
      return 0
    lax.fori_loop(0, C // 2, edge_body, 0)

    for src, dst in wdescs(k, b):
      pltpu.async_copy(src, dst, wsem[b])
    for t in GS:
      rowsc[b][pl.ds(t, L)] = rowv[b][pl.ds(t, L)]
    src, dst = sdesc(b)
    pltpu.async_copy(src, dst, ssem[b], add=True)
    @pl.when(k < kpt - 2)
    def _():
      for src, dst in idescs(k + 2, b):
        pltpu.async_copy(src, dst, isem[b])

  for src, dst in idescs(0, 0):
    pltpu.sync_copy(src, dst)
  for src, dst in idescs(1, 1):
    pltpu.sync_copy(src, dst)
  for src, dst in gdescs(0):
    pltpu.async_copy(src, dst, gsem[0])

  def pair_body(p, _):
    for b in range(2):
      process(p * 2 + b, b)
    return 0
  assert True
  lax.fori_loop(0, kpt // 2, pair_body, 0)
  if kpt % 2:
    process(kpt - 1, (kpt - 1) % 2)

  for b in range(2):
    klast = kpt - 1 - ((kpt - 1 + b) % 2)
    for src, dst in wdescs(klast, b):
      pltpu.make_async_copy(src, dst, wsem[b]).wait()
    src, dst = sdesc(b)
    pltpu.make_async_copy(src, dst, ssem[b]).wait()

  plsc.subcore_barrier()
  _per_tile_ranges(
      sid, n,
      lambda s, c: pltpu.sync_copy(aggsh.at[pl.ds(s, c)],
                                   aggp_hbm.at[cid, pl.ds(s, c)]))


def _coord_sc_body(cw_hbm, rx_hbm, ry_hbm, rz_hbm, row4_hbm,
                   cupx_hbm, cupy_hbm, cupz_hbm, cdeg_hbm,
                   rowa, cwa, rxa, rya, rza, pxa, pya, pza, onesb, zb,
                   cshx, cshy, cshz, cshd, ssem):
  n = cshx.shape[0]
  cid = lax.axis_index("c")
  sid = lax.axis_index("s")
  wid = sid * NC + cid
  kpt = rowa.shape[0]
  ept = kpt * CS
  tile_e0 = wid * ept

  one = jnp.full((L,), 1.0, jnp.float32)
  for t in range(CS // L):
    onesb[pl.ds(t * L, L)] = one
  for t in range(ZR // L):
    zb[pl.ds(t * L, L)] = jnp.zeros((L,), jnp.float32)
  for sh in (cshx, cshy, cshz, cshd):
    _per_tile_ranges(sid, n, lambda s, c, sh=sh: _fill_rows(zb, ZR, sh, s, c))

  pltpu.sync_copy(row4_hbm.at[wid], rowa)
  pltpu.sync_copy(cw_hbm.at[pl.ds(tile_e0, ept)], cwa)
  pltpu.sync_copy(rx_hbm.at[pl.ds(tile_e0, ept)], rxa)
  pltpu.sync_copy(ry_hbm.at[pl.ds(tile_e0, ept)], rya)
  pltpu.sync_copy(rz_hbm.at[pl.ds(tile_e0, ept)], rza)

  def prod_body(i, _):
    s = pl.ds(i * L, L)
    w = cwa[s]
    pxa[s] = w * rxa[s]
    pya[s] = w * rya[s]
    pza[s] = w * rza[s]
    return 0
  lax.fori_loop(0, ept // L, prod_body, 0)
  plsc.subcore_barrier()

  def sdescs(k):
    es = pl.ds(k * CS, CS)
    return [(pxa.at[es], cshx.at[rowa.at[k, 0]]),
            (pya.at[es], cshy.at[rowa.at[k, 0]]),
            (pza.at[es], cshz.at[rowa.at[k, 0]]),
            (onesb, cshd.at[rowa.at[k, 0]])]

  def chunk_body(k, _):
    for src, dst in sdescs(k):
      pltpu.async_copy(src, dst, ssem, add=True)
    @pl.when(k >= 1)
    def _():
      for src, dst in sdescs(k - 1):
        pltpu.make_async_copy(src, dst, ssem).wait()
    return 0
  lax.fori_loop(0, kpt, chunk_body, 0)
  for src, dst in sdescs(kpt - 1):
    pltpu.make_async_copy(src, dst, ssem).wait()

  plsc.subcore_barrier()
  def _staged_out(sh, out, s, c):
    for t in range(0, c, ZR):
      cc = min(ZR, c - t)
      pltpu.sync_copy(sh.at[pl.ds(s + t, cc)], pxa.at[pl.ds(0, cc)])
      pltpu.sync_copy(pxa.at[pl.ds(0, cc)],
                      out.at[pl.ds(cid * n + s + t, cc)])
  for sh, out in ((cshx, cupx_hbm), (cshy, cupy_hbm), (cshz, cupz_hbm),
                  (cshd, cdeg_hbm)):
    _per_tile_ranges(
        sid, n,
        lambda s, c, sh=sh, out=out: _staged_out(sh, out, s, c))


def kernel(h, coords, edge_index, We1, be1, We2, be2, Wn1, bn1, Wn2, bn2,
           Wc1, bc1, Wc2):
  n, d = h.shape
  e = edge_index.shape[1]
  assert e % (NW * C) == 0 and e % (NW * CS) == 0 and d % L == 0
  kpts = e // (NW * CS)

  row = edge_index[0]
  col = edge_index[1]
  row4 = row.reshape(NW, kpts, 1, CS)
  cx = coords[:, 0]
  cy = coords[:, 1]
  cz = coords[:, 2]
  we1a = We1[:, :d]
  we1b = We1[:, d:2 * d]
  w3 = We1[:, 2 * d]
  be1r = be1.reshape(1, d)
  be2r = be2.reshape(1, d)
  bn1r = bn1.reshape(1, d)
  bn2r = bn2.reshape(1, d)
  bc1r = bc1.reshape(1, d)

  bn = 1000
  gridn = n // bn
  full = lambda shape: pl.BlockSpec(shape, lambda i: (0,) * len(shape))

  a_m, b_m, wp, bp = pl.pallas_call(
      _prep_body,
      grid=(gridn,),
      in_specs=[pl.BlockSpec((bn, d), lambda i: (i, 0)),
                full((d, d)), full((d, d)), full((1, d)),
                full((d, d)), full((d, d)), full((1, d)), full((1, d))],
      out_specs=[pl.BlockSpec((bn, d), lambda i: (i, 0)),
                 pl.BlockSpec((bn, d), lambda i: (i, 0)),
                 full((d, d)), full((1, d))],
      out_shape=[jax.ShapeDtypeStruct((n, d), jnp.float32),
                 jax.ShapeDtypeStruct((n, d), jnp.float32),
                 jax.ShapeDtypeStruct((d, d), jnp.float32),
                 jax.ShapeDtypeStruct((1, d), jnp.float32)],
  )(h, we1a, we1b, be1r, We2, Wc1, be2r, bc1r)

  mesh = plsc.VectorSubcoreMesh(core_axis_name="c", subcore_axis_name="s")
  dbl = lambda shape, dt: [pltpu.VMEM(shape, dt), pltpu.VMEM(shape, dt)]
  def make_edge_sc(esz):
    return functools.partial(
        pl.kernel, mesh=mesh,
        out_type=[jax.ShapeDtypeStruct((esz, d), jnp.float32),
                  jax.ShapeDtypeStruct((esz,), jnp.float32),
                  jax.ShapeDtypeStruct((esz,), jnp.float32),
                  jax.ShapeDtypeStruct((esz,), jnp.float32),
                  jax.ShapeDtypeStruct((NC, n, d), jnp.float32)],
        scratch_types=(
            dbl((C,), jnp.int32) + dbl((C,), jnp.int32)
            + dbl((C, d), jnp.float32) + dbl((C, d), jnp.float32)
            + dbl((C,), jnp.float32) * 6
            + dbl((C, d), jnp.float32)
            + dbl((C,), jnp.float32) * 3
            + dbl((C,), jnp.int32)
            + [pltpu.VMEM((d,), jnp.float32),
               pltpu.SMEM((C,), jnp.float32)]
            + [pltpu.VMEM_SHARED((n, d), jnp.float32)]
            + [pltpu.SemaphoreType.DMA] * 8
        ))(_edge_sc_body)

  e2 = e // 2
  assert e2 % (NW * C) == 0
  edge_sc = make_edge_sc(e2)
  r_1, rx1, ry1, rz1, aggp1 = edge_sc(a_m, b_m, cx, cy, cz,
                                      row[:e2], col[:e2], w3)
  r_2, rx2, ry2, rz2, aggp2 = edge_sc(a_m, b_m, cx, cy, cz,
                                      row[e2:], col[e2:], w3)

  be = 2000
  def edge_mlp(r_h):
    return pl.pallas_call(
        _edge_mlp_body,
        grid=(e2 // be,),
        in_specs=[pl.BlockSpec((be, d), lambda i: (i, 0)),
                  full((d, d)), full((1, d)), full((1, d))],
        out_specs=pl.BlockSpec((1, 1, be), lambda i: (i, 0, 0)),
        out_shape=jax.ShapeDtypeStruct((e2 // be, 1, be), jnp.float32),
    )(r_h, wp, bp, Wc2).reshape(e2)
  cw = jnp.concatenate([edge_mlp(r_1), edge_mlp(r_2)], axis=0)
  relx = jnp.concatenate([rx1, rx2])
  rely = jnp.concatenate([ry1, ry2])
  relz = jnp.concatenate([rz1, rz2])

  coord_sc = functools.partial(
      pl.kernel, mesh=mesh,
      out_type=[jax.ShapeDtypeStruct((NC * n,), jnp.float32)] * 4,
      scratch_types=(
          [pltpu.VMEM((kpts, 1, CS), jnp.int32)]
          + [pltpu.VMEM((kpts * CS,), jnp.float32)] * 7
          + [pltpu.VMEM((CS,), jnp.float32), pltpu.VMEM((ZR,), jnp.float32)]
          + [pltpu.VMEM_SHARED((n,), jnp.float32)] * 4
          + [pltpu.SemaphoreType.DMA]
      ))(_coord_sc_body)
  cupx, cupy, cupz, cdeg = coord_sc(cw, relx, rely, relz, row4)

  cup4 = lambda a: a.reshape(NC, gridn, 1, bn)
  cup_spec = pl.BlockSpec((NC, 1, 1, bn), lambda i: (0, i, 0, 0))
  hnew, cout = pl.pallas_call(
      _node_body,
      grid=(gridn,),
      in_specs=[pl.BlockSpec((bn, d), lambda i: (i, 0)),
                pl.BlockSpec((bn, d), lambda i: (i, 0)),
                pl.BlockSpec((bn, d), lambda i: (i, 0)),
                pl.BlockSpec((bn, d), lambda i: (i, 0)),
                pl.BlockSpec((bn, d), lambda i: (i, 0)),
                cup_spec, cup_spec, cup_spec, cup_spec,
                pl.BlockSpec((bn, 3), lambda i: (i, 0)),
                full((d, d)), full((1, d)), full((d, d)), full((d, d)),
                full((1, d)), full((d, d)), full((1, d))],
      out_specs=[pl.BlockSpec((bn, d), lambda i: (i, 0)),
                 pl.BlockSpec((bn, 3), lambda i: (i, 0))],
      out_shape=[jax.ShapeDtypeStruct((n, d), jnp.float32),
                 jax.ShapeDtypeStruct((n, 3), jnp.float32)],
  )(h, aggp1[0], aggp1[1], aggp2[0], aggp2[1],
    cup4(cupx), cup4(cupy), cup4(cupz), cup4(cdeg),
    coords, We2, be2r,
    Wn1[:, :d], Wn1[:, d:], bn1r, Wn2, bn2r)

  return hnew, cout
